# baseline (speedup 1.0000x reference)
import jax
import jax.numpy as jnp
from jax import lax
from jax.experimental import pallas as pl
from jax.experimental.pallas import tpu as pltpu
from jax.experimental.pallas import tpu_sc as plsc

_L = 200
_B = 1024
_M = 64
_SUB = 128
_UB = 256
_NQ = _B // _UB
_NR = _B // _SUB
_NU = 3 * (_L - 1) * _NQ
_NW = 32


def _body(ly_h, lp_h, ry_h, lyg_h, lpg_h, ryg_h, emb_h, pos_h,
          lo_h, po_h, ro_h, ridx_v, gidx_v, wide_v, out_v, sem_g, sem_s):
    c = lax.axis_index("c")
    s = lax.axis_index("s")
    w = s * 2 + c

    iota = lax.iota(jnp.int32, 16)
    zvec = jnp.zeros((16,), jnp.float32)

    def _zrow(m, carry):
        for cc in range(_SUB // 16):
            out_v[0, m, pl.ds(cc * 16, 16)] = zvec
        return carry

    lax.fori_loop(0, _M, _zrow, 0)
    zq = pl.multiple_of((w % 8) * _SUB, _SUB)

    @pl.when(w < 8)
    def _():
        pltpu.sync_copy(out_v.at[0, :, pl.ds(0, _SUB)],
                        lo_h.at[0, :, pl.ds(zq, _SUB)])

    @pl.when(jnp.logical_and(w >= 8, w < 16))
    def _():
        pltpu.sync_copy(out_v.at[0, :, pl.ds(0, _SUB)],
                        po_h.at[0, :, pl.ds(zq, _SUB)])

    @pl.when(jnp.logical_and(w >= 16, w < 24))
    def _():
        pltpu.sync_copy(out_v.at[0, :, pl.ds(0, _SUB)],
                        ro_h.at[0, :, pl.ds(zq, _SUB)])

    nu = (_NU // _NW) + jnp.where(w < (_NU % _NW), 1, 0)

    def _split(uid):
        task = uid % 3
        rem = uid // 3
        l = 1 + rem // _NQ
        q = rem % _NQ
        return task, l, q

    def _fire(uid, b):
        task, l, q = _split(uid)
        row_s = pl.multiple_of((l - 1) * _NR, _NR)
        row_r = pl.multiple_of(l * _NR, _NR)

        def _one(r_h, g_h, tab_h, row):
            pltpu.sync_copy(r_h.at[pl.ds(row, _NR)], ridx_v.at[b])
            pltpu.sync_copy(g_h.at[pl.ds(row, _NR)], gidx_v.at[b])
            for j in range(_UB // _SUB):
                pltpu.async_copy(tab_h.at[gidx_v.at[b, q * 2 + j]],
                                 wide_v.at[b, pl.ds(j * _SUB, _SUB)], sem_g)

        @pl.when(task == 0)
        def _():
            _one(ly_h, lyg_h, emb_h, row_s)

        @pl.when(task == 1)
        def _():
            _one(lp_h, lpg_h, pos_h, row_s)

        @pl.when(task == 2)
        def _():
            _one(ry_h, ryg_h, emb_h, row_r)

    def _wait_g(uid, b):
        _, _, q = _split(uid)
        for j in range(_UB // _SUB):
            pltpu.make_async_copy(emb_h.at[gidx_v.at[b, q * 2 + j]],
                                  wide_v.at[b, pl.ds(j * _SUB, _SUB)],
                                  sem_g).wait()

    def _extract(uid, b):
        _, _, q = _split(uid)
        bsplat = jnp.full((16,), b, jnp.int32)

        def _grp(g, carry):
            rvec = ridx_v[b, q * 2 + g // 8, pl.ds((g % 8) * 16, 16)]
            bvec = g * 16 + iota
            cvec = (rvec & 1) * 64
            for m in range(_M):
                val = plsc.load_gather(wide_v, [bsplat, bvec, cvec + m])
                out_v[b, m, pl.ds(g * 16, 16)] = val
            return carry

        lax.fori_loop(0, _UB // 16, _grp, 0)

    def _store(uid, b):
        task, l, q = _split(uid)
        off = pl.multiple_of(q * _UB, _UB)

        @pl.when(task == 0)
        def _():
            pltpu.async_copy(out_v.at[b],
                             lo_h.at[l, :, pl.ds(off, _UB)], sem_s)

        @pl.when(task == 1)
        def _():
            pltpu.async_copy(out_v.at[b],
                             po_h.at[l, :, pl.ds(off, _UB)], sem_s)

        @pl.when(task == 2)
        def _():
            pltpu.async_copy(out_v.at[b],
                             ro_h.at[l, :, pl.ds(off, _UB)], sem_s)

    def _wait_s():
        pltpu.make_async_copy(lo_h.at[0, :, pl.ds(0, _UB)], out_v.at[0],
                              sem_s).wait()

    _fire(w, 0)

    def _step(p, carry):
        i0 = 2 * p
        i1 = i0 + 1
        u0 = w + i0 * _NW
        u1 = w + i1 * _NW

        @pl.when(i1 < nu)
        def _():
            _fire(u1, 1)

        _wait_g(u0, 0)

        @pl.when(i0 >= 2)
        def _():
            _wait_s()

        _extract(u0, 0)
        _store(u0, 0)

        @pl.when(i1 < nu)
        def _():
            @pl.when(i1 + 1 < nu)
            def _():
                _fire(u1 + _NW, 0)

            _wait_g(u1, 1)

            @pl.when(i1 >= 2)
            def _():
                _wait_s()

            _extract(u1, 1)
            _store(u1, 1)

        return carry

    lax.fori_loop(0, (nu + 1) // 2, _step, 0)
    _wait_s()
    _wait_s()


@jax.jit
def kernel(ly, lp, ry, emb_table, pos_table):
    nr = _L * _B // _SUB
    ly2 = ly.astype(jnp.int32).reshape(nr, _SUB)
    lp2 = lp.astype(jnp.int32).reshape(nr, _SUB)
    ry2 = ry.astype(jnp.int32).reshape(nr, _SUB)
    lyg = (ly2 >> 1)
    lpg = (lp2 >> 1)
    ryg = (ry2 >> 1)
    emb_w = emb_table.reshape(emb_table.shape[0] // 2, 2 * _M)
    pos_w = pos_table.reshape(pos_table.shape[0] // 2, 2 * _M)

    mesh = plsc.VectorSubcoreMesh(core_axis_name="c", subcore_axis_name="s")
    out3 = (jax.ShapeDtypeStruct((_L, _M, _B), jnp.float32),) * 3
    run = pl.kernel(
        _body,
        mesh=mesh,
        out_type=out3,
        scratch_types=[
            pltpu.VMEM((2, _NR, _SUB), jnp.int32),
            pltpu.VMEM((2, _NR, _SUB), jnp.int32),
            pltpu.VMEM((2, _UB, 2 * _M), jnp.float32),
            pltpu.VMEM((2, _M, _UB), jnp.float32),
            pltpu.SemaphoreType.DMA,
            pltpu.SemaphoreType.DMA,
        ],
        compiler_params=pltpu.CompilerParams(needs_layout_passes=False),
    )
    lo, po, ro = run(ly2, lp2, ry2, lyg, lpg, ryg, emb_w, pos_w)
    return (jnp.transpose(lo, (0, 2, 1)),
            jnp.transpose(po, (0, 2, 1)),
            jnp.transpose(ro, (0, 2, 1)))
